# Initial kernel scaffold; baseline (speedup 1.0000x reference)
import functools

import jax
import jax.numpy as jnp
from jax import lax
from jax.experimental import pallas as pl
from jax.experimental.pallas import tpu as pltpu
from jax.experimental.pallas import tpu_sc as plsc

N = 50000
E = 800000
D = 64
NC = 2
NS = 16
L = 16

HALF = N // NC
TROWS = 25088
RPT = TROWS // NS
LASTR = HALF - (NS - 1) * RPT
TRASH = HALF

B = 128
EPT = E // NS
NFULL = EPT // B
TAIL = EPT - NFULL * B

_mesh = plsc.VectorSubcoreMesh(core_axis_name="c", subcore_axis_name="s")


@functools.partial(
    pl.kernel,
    out_type=jax.ShapeDtypeStruct((N, D), jnp.float32),
    mesh=_mesh,
    scratch_types=[
        pltpu.VMEM_SHARED((TROWS, D), jnp.float32),
        pltpu.VMEM((B,), jnp.int32),
        pltpu.VMEM((B,), jnp.int32),
        pltpu.VMEM((B,), jnp.int32),
        pltpu.VMEM((B, D), jnp.float32),
        pltpu.VMEM((TAIL,), jnp.int32),
        pltpu.VMEM((TAIL,), jnp.int32),
        pltpu.VMEM((TAIL,), jnp.int32),
        pltpu.VMEM((TAIL, D), jnp.float32),
        pltpu.SemaphoreType.DMA,
    ],
)
def _sc_agg(h_hbm, src_hbm, dst_hbm, zeros_hbm, out_hbm,
            table, srcv, dstv, dstl, rows,
            srcv_t, dstv_t, dstl_t, rows_t, sem):
    c = lax.axis_index("c")
    s = lax.axis_index("s")
    base = c * HALF

    pltpu.sync_copy(zeros_hbm, table.at[pl.ds(s * RPT, RPT)])
    plsc.subcore_barrier()

    def remap(dv, dlv, nb):
        for j in range(nb // L):
            d = dv[pl.ds(j * L, L)]
            m = (d >= base) & (d < base + HALF)
            dlv[pl.ds(j * L, L)] = jnp.where(m, d - base, TRASH)

    def body(i, carry):
        e0 = pl.multiple_of(s * EPT + i * B, 8)
        pltpu.sync_copy(src_hbm.at[pl.ds(e0, B)], srcv)
        pltpu.sync_copy(dst_hbm.at[pl.ds(e0, B)], dstv)
        remap(dstv, dstl, B)
        pltpu.async_copy(h_hbm.at[srcv], rows, sem).wait()
        pltpu.sync_copy(rows, table.at[dstl], add=True)
        return carry

    lax.fori_loop(0, NFULL, body, 0)

    if TAIL:
        e0 = pl.multiple_of(s * EPT + NFULL * B, 8)
        pltpu.sync_copy(src_hbm.at[pl.ds(e0, TAIL)], srcv_t)
        pltpu.sync_copy(dst_hbm.at[pl.ds(e0, TAIL)], dstv_t)
        remap(dstv_t, dstl_t, TAIL)
        pltpu.async_copy(h_hbm.at[srcv_t], rows_t, sem).wait()
        pltpu.sync_copy(rows_t, table.at[dstl_t], add=True)

    plsc.subcore_barrier()

    @pl.when(s < NS - 1)
    def _():
        pltpu.sync_copy(table.at[pl.ds(s * RPT, RPT)],
                        out_hbm.at[pl.ds(base + s * RPT, RPT)])

    @pl.when(s == NS - 1)
    def _():
        pltpu.sync_copy(table.at[pl.ds(s * RPT, LASTR)],
                        out_hbm.at[pl.ds(base + s * RPT, LASTR)])


BN = 1024


def _mlp_body(x_ref, agg_ref, w1_ref, b1_ref, w2_ref, b2_ref, out_ref):
    h = x_ref[...] + agg_ref[...]
    h = jnp.tanh(jnp.dot(h, w1_ref[...], preferred_element_type=jnp.float32)
                 + b1_ref[...])
    out_ref[...] = (jnp.dot(h, w2_ref[...], preferred_element_type=jnp.float32)
                    + b2_ref[...])


def _mlp(x, agg, w1, b1, w2, b2):
    full = lambda i: (0, 0)
    return pl.pallas_call(
        _mlp_body,
        grid=(pl.cdiv(N, BN),),
        in_specs=[
            pl.BlockSpec((BN, D), lambda i: (i, 0)),
            pl.BlockSpec((BN, D), lambda i: (i, 0)),
            pl.BlockSpec((D, D), full),
            pl.BlockSpec((1, D), full),
            pl.BlockSpec((D, D), full),
            pl.BlockSpec((1, D), full),
        ],
        out_specs=pl.BlockSpec((BN, D), lambda i: (i, 0)),
        out_shape=jax.ShapeDtypeStruct((N, D), jnp.float32),
    )(x, agg, w1, b1, w2, b2)


def kernel(x, edge_index, W1_0, b1_0, W2_0, b2_0, W1_1, b1_1, W2_1, b2_1):
    src = edge_index[0].astype(jnp.int32)
    dst = edge_index[1].astype(jnp.int32)
    zeros = jnp.zeros((RPT, D), jnp.float32)
    h = x
    for (w1, b1, w2, b2) in ((W1_0, b1_0, W2_0, b2_0),
                             (W1_1, b1_1, W2_1, b2_1)):
        agg = _sc_agg(h, src, dst, zeros)
        h = _mlp(h, agg, w1, b1.reshape(1, D), w2, b2.reshape(1, D))
    return h


# speedup vs baseline: 3.6421x; 3.6421x over previous
import functools

import jax
import jax.numpy as jnp
from jax import lax
from jax.experimental import pallas as pl
from jax.experimental.pallas import tpu as pltpu
from jax.experimental.pallas import tpu_sc as plsc

N = 50000
E = 800000
D = 64
NC = 2
NS = 16
L = 16

HALF = N // NC
TROWS = 25088
RPT = TROWS // NS
LASTR = HALF - (NS - 1) * RPT
TRASH = HALF

B = 128
EPT = E // NS
NFULL = EPT // B
TAIL = EPT - NFULL * B

_mesh = plsc.VectorSubcoreMesh(core_axis_name="c", subcore_axis_name="s")


@functools.partial(
    pl.kernel,
    out_type=jax.ShapeDtypeStruct((N, D), jnp.float32),
    mesh=_mesh,
    compiler_params=pltpu.CompilerParams(use_tc_tiling_on_sc=False),
    scratch_types=[
        pltpu.VMEM_SHARED((TROWS, D), jnp.float32),
        pltpu.VMEM((B,), jnp.int32),
        pltpu.VMEM((B,), jnp.int32),
        pltpu.VMEM((B,), jnp.int32),
        pltpu.VMEM((B, D), jnp.float32),
        pltpu.VMEM((TAIL,), jnp.int32),
        pltpu.VMEM((TAIL,), jnp.int32),
        pltpu.VMEM((TAIL,), jnp.int32),
        pltpu.VMEM((TAIL, D), jnp.float32),
        pltpu.SemaphoreType.DMA,
    ],
)
def _sc_agg(h_hbm, src_hbm, dst_hbm, zeros_hbm, out_hbm,
            table, srcv, dstv, dstl, rows,
            srcv_t, dstv_t, dstl_t, rows_t, sem):
    c = lax.axis_index("c")
    s = lax.axis_index("s")
    base = c * HALF

    pltpu.sync_copy(zeros_hbm, table.at[pl.ds(s * RPT, RPT)])
    plsc.subcore_barrier()

    def remap(dv, dlv, nb):
        for j in range(nb // L):
            d = dv[pl.ds(j * L, L)]
            m = (d >= base) & (d < base + HALF)
            dlv[pl.ds(j * L, L)] = jnp.where(m, d - base, TRASH)

    def body(i, carry):
        e0 = pl.multiple_of(s * EPT + i * B, 8)
        pltpu.sync_copy(src_hbm.at[pl.ds(e0, B)], srcv)
        pltpu.sync_copy(dst_hbm.at[pl.ds(e0, B)], dstv)
        remap(dstv, dstl, B)
        pltpu.async_copy(h_hbm.at[srcv], rows, sem).wait()
        pltpu.sync_copy(rows, table.at[dstl], add=True)
        return carry

    lax.fori_loop(0, NFULL, body, 0)

    if TAIL:
        e0 = pl.multiple_of(s * EPT + NFULL * B, 8)
        pltpu.sync_copy(src_hbm.at[pl.ds(e0, TAIL)], srcv_t)
        pltpu.sync_copy(dst_hbm.at[pl.ds(e0, TAIL)], dstv_t)
        remap(dstv_t, dstl_t, TAIL)
        pltpu.async_copy(h_hbm.at[srcv_t], rows_t, sem).wait()
        pltpu.sync_copy(rows_t, table.at[dstl_t], add=True)

    plsc.subcore_barrier()

    @pl.when(s < NS - 1)
    def _():
        pltpu.sync_copy(table.at[pl.ds(s * RPT, RPT)],
                        out_hbm.at[pl.ds(base + s * RPT, RPT)])

    @pl.when(s == NS - 1)
    def _():
        pltpu.sync_copy(table.at[pl.ds(s * RPT, LASTR)],
                        out_hbm.at[pl.ds(base + s * RPT, LASTR)])


BN = 1024


def _mlp_body(x_ref, agg_ref, w1_ref, b1_ref, w2_ref, b2_ref, out_ref):
    h = x_ref[...] + agg_ref[...]
    h = jnp.tanh(jnp.dot(h, w1_ref[...], preferred_element_type=jnp.float32)
                 + b1_ref[...])
    out_ref[...] = (jnp.dot(h, w2_ref[...], preferred_element_type=jnp.float32)
                    + b2_ref[...])


def _mlp(x, agg, w1, b1, w2, b2):
    full = lambda i: (0, 0)
    return pl.pallas_call(
        _mlp_body,
        grid=(pl.cdiv(N, BN),),
        in_specs=[
            pl.BlockSpec((BN, D), lambda i: (i, 0)),
            pl.BlockSpec((BN, D), lambda i: (i, 0)),
            pl.BlockSpec((D, D), full),
            pl.BlockSpec((1, D), full),
            pl.BlockSpec((D, D), full),
            pl.BlockSpec((1, D), full),
        ],
        out_specs=pl.BlockSpec((BN, D), lambda i: (i, 0)),
        out_shape=jax.ShapeDtypeStruct((N, D), jnp.float32),
    )(x, agg, w1, b1, w2, b2)


def kernel(x, edge_index, W1_0, b1_0, W2_0, b2_0, W1_1, b1_1, W2_1, b2_1):
    src = edge_index[0].astype(jnp.int32)
    dst = edge_index[1].astype(jnp.int32)
    zeros = jnp.zeros((RPT, D), jnp.float32)
    h = x
    for (w1, b1, w2, b2) in ((W1_0, b1_0, W2_0, b2_0),
                             (W1_1, b1_1, W2_1, b2_1)):
        agg = _sc_agg(h, src, dst, zeros)
        h = _mlp(h, agg, w1, b1.reshape(1, D), w2, b2.reshape(1, D))
    return h
